# baseline (speedup 1.0000x reference)
import functools

import jax
import jax.numpy as jnp
import numpy as np
from jax import lax
from jax.experimental import pallas as pl
from jax.experimental.pallas import tpu as pltpu
from jax.experimental.pallas import tpu_sc as plsc

N_NODES = 10000
N_EDGES = 160000
MUL = 16
IN_DIM = 64
N_EDGE_FEAT = 256
HIDDEN = 256
WEIGHT_NUMEL = 1024

NC = 2
NS = 16
NW = NC * NS
EPW = N_EDGES // NW
CH = 128
NCHUNK = (EPW + CH - 1) // CH
TAIL_OFF = EPW - CH
TAIL_NEW = EPW - (NCHUNK - 1) * CH
NACC = 10112
STRIPE = NACC // NS
TRASH = N_NODES

BE = 256
NBLK = N_EDGES // BE

_INV_SQRT3 = 1.0 / np.sqrt(3.0)
_INV_SQRT32 = 1.0 / np.sqrt(32.0)


def _gather_body(table_hbm, idx_hbm, out_hbm, idx_v, rows_v, sem):
    c = lax.axis_index("c")
    s = lax.axis_index("s")
    wid = c * NS + s
    pltpu.sync_copy(idx_hbm.at[wid], idx_v)

    def body(j, carry):
        off = wid * EPW + jnp.minimum(j * CH, TAIL_OFF)
        pltpu.async_copy(table_hbm.at[idx_v.at[j]], rows_v, sem).wait()
        pltpu.sync_copy(rows_v, out_hbm.at[pl.ds(off, CH)])
        return carry

    lax.fori_loop(0, NCHUNK, body, 0)


@functools.lru_cache(maxsize=None)
def _gather_call():
    return functools.partial(
        pl.kernel,
        _gather_body,
        out_type=jax.ShapeDtypeStruct((N_EDGES, 128), jnp.float32),
        mesh=plsc.VectorSubcoreMesh(core_axis_name="c", subcore_axis_name="s"),
        scratch_types=[
            pltpu.VMEM((NCHUNK, CH), jnp.int32),
            pltpu.VMEM((CH, 128), jnp.float32),
            pltpu.SemaphoreType.DMA,
        ],
    )()


def _scatter_body(tp_hbm, idx_hbm, zeros_hbm, out_hbm, idx_v, rows_v, acc, sem):
    c = lax.axis_index("c")
    s = lax.axis_index("s")
    wid = c * NS + s
    pltpu.sync_copy(idx_hbm.at[wid], idx_v)

    @pl.when(s == 0)
    def _():
        pltpu.sync_copy(zeros_hbm, acc)

    plsc.subcore_barrier()

    def body(j, carry):
        off = wid * EPW + jnp.minimum(j * CH, TAIL_OFF)
        pltpu.async_copy(tp_hbm.at[pl.ds(off, CH)], rows_v, sem).wait()
        pltpu.sync_copy(rows_v, acc.at[idx_v.at[j]], add=True)
        return carry

    lax.fori_loop(0, NCHUNK, body, 0)
    plsc.subcore_barrier()
    pltpu.sync_copy(acc.at[pl.ds(s * STRIPE, STRIPE)],
                    out_hbm.at[c].at[pl.ds(s * STRIPE, STRIPE)])


@functools.lru_cache(maxsize=None)
def _scatter_call():
    return pl.kernel(
        _scatter_body,
        out_type=jax.ShapeDtypeStruct((NC, NACC, 128), jnp.float32),
        mesh=plsc.VectorSubcoreMesh(core_axis_name="c", subcore_axis_name="s"),
        scratch_types=[
            pltpu.VMEM((NCHUNK, CH), jnp.int32),
            pltpu.VMEM((CH, 128), jnp.float32),
            pltpu.VMEM_SHARED((NACC, 128), jnp.float32),
            pltpu.SemaphoreType.DMA,
        ],
    )


def _tc_body(ea_ref, g_ref, sh_ref, ab_ref, w1_ref, b1_ref, w2_ref, b2_ref,
             red_ref, out_ref):
    x = ea_ref[...]
    h = jnp.maximum(
        jnp.dot(x, w1_ref[...], preferred_element_type=jnp.float32) + b1_ref[...], 0.0)
    tpw = jnp.dot(h, w2_ref[...], preferred_element_type=jnp.float32) + b2_ref[...]

    shab = jnp.dot(sh_ref[...], ab_ref[...], preferred_element_type=jnp.float32)

    g = g_ref[...]
    m = g * shab[:, 0:128]
    n = g * shab[:, 128:256]
    dot = m[:, 112:128] + n[:, 48:64] + n[:, 80:96]
    u0 = jnp.concatenate([m[:, 96:112], dot], 1)
    ux = m[:, 0:32]
    uy = m[:, 32:64]
    uz = m[:, 64:96]

    t0 = jnp.concatenate([u0] * 16, 1)
    tx = jnp.concatenate([ux] * 16, 1)
    ty = jnp.concatenate([uy] * 16, 1)
    tz = jnp.concatenate([uz] * 16, 1)
    w0 = tpw[:, 0:512]
    w1 = tpw[:, 512:1024]
    pall = jnp.concatenate([t0 * w0, tx * w1, ty * w1, tz * w1], 0)

    sred = jnp.dot(pall, red_ref[...], preferred_element_type=jnp.float32)

    ones_col = jnp.concatenate(
        [jnp.ones((BE, 16), jnp.float32), jnp.zeros((BE, 48), jnp.float32)], 1)
    out_ref[...] = jnp.concatenate(
        [sred[0:BE], sred[BE:2 * BE], sred[2 * BE:3 * BE], sred[3 * BE:4 * BE],
         ones_col], 1)


def _tc_fused(edge_attr, g, edge_sh, ab, w1, b1, w2p, b2p, red):
    return pl.pallas_call(
        _tc_body,
        grid=(NBLK,),
        in_specs=[
            pl.BlockSpec((BE, N_EDGE_FEAT), lambda i: (i, 0)),
            pl.BlockSpec((BE, 128), lambda i: (i, 0)),
            pl.BlockSpec((BE, 4), lambda i: (i, 0)),
            pl.BlockSpec((4, 256), lambda i: (0, 0)),
            pl.BlockSpec((N_EDGE_FEAT, HIDDEN), lambda i: (0, 0)),
            pl.BlockSpec((1, HIDDEN), lambda i: (0, 0)),
            pl.BlockSpec((HIDDEN, WEIGHT_NUMEL), lambda i: (0, 0)),
            pl.BlockSpec((1, WEIGHT_NUMEL), lambda i: (0, 0)),
            pl.BlockSpec((512, 16), lambda i: (0, 0)),
        ],
        out_specs=pl.BlockSpec((BE, 128), lambda i: (i, 0)),
        out_shape=jax.ShapeDtypeStruct((N_EDGES, 128), jnp.float32),
        compiler_params=pltpu.CompilerParams(
            dimension_semantics=("arbitrary",)),
    )(edge_attr, g, edge_sh, ab, w1, b1, w2p, b2p, red)


def _fin_body(a_ref, b_ref, n_ref, out_ref):
    a = a_ref[...]
    b = b_ref[...]
    tot = a[:, 0:64] + b[:, 0:64]
    cnt = a[:, 64:65] + b[:, 64:65]
    out_ref[...] = tot / jnp.maximum(cnt, 1.0) + n_ref[...]


def _finalize(acc0, acc1, node_planes):
    return pl.pallas_call(
        _fin_body,
        grid=(10,),
        in_specs=[
            pl.BlockSpec((1000, 128), lambda i: (i, 0)),
            pl.BlockSpec((1000, 128), lambda i: (i, 0)),
            pl.BlockSpec((1000, IN_DIM), lambda i: (i, 0)),
        ],
        out_specs=pl.BlockSpec((1000, IN_DIM), lambda i: (i, 0)),
        out_shape=jax.ShapeDtypeStruct((N_NODES, IN_DIM), jnp.float32),
    )(acc0, acc1, node_planes)


_Q = np.arange(512)
_PERM = np.concatenate([16 * (_Q % 32) + _Q // 32,
                        512 + 16 * (_Q % 32) + _Q // 32]).astype(np.int32)

_RED = np.zeros((512, 16), np.float32)
_RED[_Q, _Q // 32] = _INV_SQRT32

_AB = np.zeros((4, 256), np.float32)
for _g, (_s, _v) in enumerate([(1, 1.0), (0, 1.0), (2, 1.0), (0, 1.0),
                               (3, 1.0), (0, 1.0), (0, 1.0), (1, _INV_SQRT3)]):
    _AB[_s, 16 * _g:16 * (_g + 1)] = _v
_AB[2, 128 + 48:128 + 64] = _INV_SQRT3
_AB[3, 128 + 80:128 + 96] = _INV_SQRT3

_OFFS = np.minimum(np.arange(NCHUNK) * CH, TAIL_OFF)
_POS = (np.arange(NW)[:, None, None] * EPW
        + _OFFS[None, :, None]
        + np.arange(CH)[None, None, :]).astype(np.int32)


def kernel(node_attr, edge_index, edge_attr, edge_sh, fc_w1, fc_b1, fc_w2, fc_b2):
    src = edge_index[0]
    dst = edge_index[1]

    n1 = node_attr[:, 16:].reshape(N_NODES, MUL, 3)
    node_planes = jnp.concatenate(
        [node_attr[:, 0:16], n1[:, :, 0], n1[:, :, 1], n1[:, :, 2]], axis=1)
    n0 = node_attr[:, 0:16]
    table = jnp.concatenate(
        [n0, n1[:, :, 0], n0, n1[:, :, 1], n0, n1[:, :, 2], n0, n1[:, :, 0]],
        axis=1)


    w2p = fc_w2[:, _PERM]
    b2p = fc_b2[_PERM].reshape(1, WEIGHT_NUMEL)
    b1 = fc_b1.reshape(1, HIDDEN)

    gidx = dst[_POS]
    sidx = src[_POS]
    sidx = sidx.at[:, NCHUNK - 1, 0:CH - TAIL_NEW].set(TRASH)

    g = _gather_call()(table, gidx)
    tp_ext = _tc_fused(edge_attr, g, edge_sh, jnp.asarray(_AB), fc_w1, b1,
                       w2p, b2p, jnp.asarray(_RED))
    zeros = jnp.zeros((NACC, 128), jnp.float32)
    acc = _scatter_call()(tp_ext, sidx, zeros)
    planes_out = _finalize(acc[0, :N_NODES], acc[1, :N_NODES], node_planes)

    o1 = planes_out[:, 16:].reshape(N_NODES, 3, MUL).transpose(0, 2, 1)
    return jnp.concatenate(
        [planes_out[:, 0:16], o1.reshape(N_NODES, 48)], axis=1)


# speedup vs baseline: 1.3520x; 1.3520x over previous
import functools

import jax
import jax.numpy as jnp
import numpy as np
from jax import lax
from jax.experimental import pallas as pl
from jax.experimental.pallas import tpu as pltpu
from jax.experimental.pallas import tpu_sc as plsc

N_NODES = 10000
N_EDGES = 160000
MUL = 16
IN_DIM = 64
N_EDGE_FEAT = 256
HIDDEN = 256
WEIGHT_NUMEL = 1024

NC = 2
NS = 16
NW = NC * NS
EPW = N_EDGES // NW
CH = 128
NCHUNK = (EPW + CH - 1) // CH
TAIL_OFF = EPW - CH
TAIL_NEW = EPW - (NCHUNK - 1) * CH
NACC = 10112
STRIPE = NACC // NS
TRASH = N_NODES

BE = 640
NBLK = N_EDGES // BE

_INV_SQRT3 = 1.0 / np.sqrt(3.0)
_INV_SQRT32 = 1.0 / np.sqrt(32.0)


def _gather_body(table_hbm, idx_hbm, out_hbm, idx_v, rows_v, sem):
    c = lax.axis_index("c")
    s = lax.axis_index("s")
    wid = c * NS + s
    pltpu.sync_copy(idx_hbm.at[wid], idx_v)

    def body(j, carry):
        off = wid * EPW + jnp.minimum(j * CH, TAIL_OFF)
        pltpu.async_copy(table_hbm.at[idx_v.at[j]], rows_v, sem).wait()
        pltpu.sync_copy(rows_v, out_hbm.at[pl.ds(off, CH)])
        return carry

    lax.fori_loop(0, NCHUNK, body, 0)


@functools.lru_cache(maxsize=None)
def _gather_call():
    return functools.partial(
        pl.kernel,
        _gather_body,
        out_type=jax.ShapeDtypeStruct((N_EDGES, 128), jnp.float32),
        mesh=plsc.VectorSubcoreMesh(core_axis_name="c", subcore_axis_name="s"),
        scratch_types=[
            pltpu.VMEM((NCHUNK, CH), jnp.int32),
            pltpu.VMEM((CH, 128), jnp.float32),
            pltpu.SemaphoreType.DMA,
        ],
    )()


def _scatter_body(tp_hbm, idx_hbm, zeros_hbm, out_hbm, idx_v, rows_v, acc, sem):
    c = lax.axis_index("c")
    s = lax.axis_index("s")
    wid = c * NS + s
    pltpu.sync_copy(idx_hbm.at[wid], idx_v)

    @pl.when(s == 0)
    def _():
        pltpu.sync_copy(zeros_hbm, acc)

    plsc.subcore_barrier()

    def body(j, carry):
        off = wid * EPW + jnp.minimum(j * CH, TAIL_OFF)
        pltpu.async_copy(tp_hbm.at[pl.ds(off, CH)], rows_v, sem).wait()
        pltpu.sync_copy(rows_v, acc.at[idx_v.at[j]], add=True)
        return carry

    lax.fori_loop(0, NCHUNK, body, 0)
    plsc.subcore_barrier()
    pltpu.sync_copy(acc.at[pl.ds(s * STRIPE, STRIPE)],
                    out_hbm.at[c].at[pl.ds(s * STRIPE, STRIPE)])


@functools.lru_cache(maxsize=None)
def _scatter_call():
    return pl.kernel(
        _scatter_body,
        out_type=jax.ShapeDtypeStruct((NC, NACC, 128), jnp.float32),
        mesh=plsc.VectorSubcoreMesh(core_axis_name="c", subcore_axis_name="s"),
        scratch_types=[
            pltpu.VMEM((NCHUNK, CH), jnp.int32),
            pltpu.VMEM((CH, 128), jnp.float32),
            pltpu.VMEM_SHARED((NACC, 128), jnp.float32),
            pltpu.SemaphoreType.DMA,
        ],
    )


def _tc_body(ea_ref, g_ref, sh_ref, ab_ref, w1_ref, b1_ref, w2_ref, b2_ref,
             red_ref, out_ref):
    x = ea_ref[...]
    h = jnp.maximum(
        jnp.dot(x, w1_ref[...], preferred_element_type=jnp.float32) + b1_ref[...], 0.0)
    tpw = jnp.dot(h, w2_ref[...], preferred_element_type=jnp.float32) + b2_ref[...]

    shab = jnp.dot(sh_ref[...], ab_ref[...], preferred_element_type=jnp.float32)

    g = g_ref[...]
    m = g * shab[:, 0:128]
    n = g * shab[:, 128:256]
    dot = m[:, 112:128] + n[:, 48:64] + n[:, 80:96]
    u0 = jnp.concatenate([m[:, 96:112], dot], 1)
    ux = m[:, 0:32]
    uy = m[:, 32:64]
    uz = m[:, 64:96]

    t0 = jnp.concatenate([u0] * 16, 1)
    tx = jnp.concatenate([ux] * 16, 1)
    ty = jnp.concatenate([uy] * 16, 1)
    tz = jnp.concatenate([uz] * 16, 1)
    w0 = tpw[:, 0:512]
    w1 = tpw[:, 512:1024]
    pall = jnp.concatenate([t0 * w0, tx * w1, ty * w1, tz * w1], 0)

    sred = jnp.dot(pall, red_ref[...], preferred_element_type=jnp.float32)

    ones_col = jnp.concatenate(
        [jnp.ones((BE, 16), jnp.float32), jnp.zeros((BE, 48), jnp.float32)], 1)
    out_ref[...] = jnp.concatenate(
        [sred[0:BE], sred[BE:2 * BE], sred[2 * BE:3 * BE], sred[3 * BE:4 * BE],
         ones_col], 1)


def _tc_fused(edge_attr, g, edge_sh, ab, w1, b1, w2p, b2p, red):
    return pl.pallas_call(
        _tc_body,
        grid=(NBLK,),
        in_specs=[
            pl.BlockSpec((BE, N_EDGE_FEAT), lambda i: (i, 0)),
            pl.BlockSpec((BE, 128), lambda i: (i, 0)),
            pl.BlockSpec((BE, 4), lambda i: (i, 0)),
            pl.BlockSpec((4, 256), lambda i: (0, 0)),
            pl.BlockSpec((N_EDGE_FEAT, HIDDEN), lambda i: (0, 0)),
            pl.BlockSpec((1, HIDDEN), lambda i: (0, 0)),
            pl.BlockSpec((HIDDEN, WEIGHT_NUMEL), lambda i: (0, 0)),
            pl.BlockSpec((1, WEIGHT_NUMEL), lambda i: (0, 0)),
            pl.BlockSpec((512, 16), lambda i: (0, 0)),
        ],
        out_specs=pl.BlockSpec((BE, 128), lambda i: (i, 0)),
        out_shape=jax.ShapeDtypeStruct((N_EDGES, 128), jnp.float32),
        compiler_params=pltpu.CompilerParams(
            dimension_semantics=("parallel",)),
    )(edge_attr, g, edge_sh, ab, w1, b1, w2p, b2p, red)


def _fin_body(a_ref, b_ref, n_ref, out_ref):
    a = a_ref[...]
    b = b_ref[...]
    tot = a[:, 0:64] + b[:, 0:64]
    cnt = a[:, 64:65] + b[:, 64:65]
    out_ref[...] = tot / jnp.maximum(cnt, 1.0) + n_ref[...]


def _finalize(acc0, acc1, node_planes):
    return pl.pallas_call(
        _fin_body,
        grid=(10,),
        in_specs=[
            pl.BlockSpec((1000, 128), lambda i: (i, 0)),
            pl.BlockSpec((1000, 128), lambda i: (i, 0)),
            pl.BlockSpec((1000, IN_DIM), lambda i: (i, 0)),
        ],
        out_specs=pl.BlockSpec((1000, IN_DIM), lambda i: (i, 0)),
        out_shape=jax.ShapeDtypeStruct((N_NODES, IN_DIM), jnp.float32),
    )(acc0, acc1, node_planes)


_Q = np.arange(512)
_PERM = np.concatenate([16 * (_Q % 32) + _Q // 32,
                        512 + 16 * (_Q % 32) + _Q // 32]).astype(np.int32)

_RED = np.zeros((512, 16), np.float32)
_RED[_Q, _Q // 32] = _INV_SQRT32

_AB = np.zeros((4, 256), np.float32)
for _g, (_s, _v) in enumerate([(1, 1.0), (0, 1.0), (2, 1.0), (0, 1.0),
                               (3, 1.0), (0, 1.0), (0, 1.0), (1, _INV_SQRT3)]):
    _AB[_s, 16 * _g:16 * (_g + 1)] = _v
_AB[2, 128 + 48:128 + 64] = _INV_SQRT3
_AB[3, 128 + 80:128 + 96] = _INV_SQRT3

_OFFS = np.minimum(np.arange(NCHUNK) * CH, TAIL_OFF)
_POS = (np.arange(NW)[:, None, None] * EPW
        + _OFFS[None, :, None]
        + np.arange(CH)[None, None, :]).astype(np.int32)


def kernel(node_attr, edge_index, edge_attr, edge_sh, fc_w1, fc_b1, fc_w2, fc_b2):
    src = edge_index[0]
    dst = edge_index[1]

    n1 = node_attr[:, 16:].reshape(N_NODES, MUL, 3)
    node_planes = jnp.concatenate(
        [node_attr[:, 0:16], n1[:, :, 0], n1[:, :, 1], n1[:, :, 2]], axis=1)
    n0 = node_attr[:, 0:16]
    table = jnp.concatenate(
        [n0, n1[:, :, 0], n0, n1[:, :, 1], n0, n1[:, :, 2], n0, n1[:, :, 0]],
        axis=1)


    w2p = fc_w2[:, _PERM]
    b2p = fc_b2[_PERM].reshape(1, WEIGHT_NUMEL)
    b1 = fc_b1.reshape(1, HIDDEN)

    gidx = dst[_POS]
    sidx = src[_POS]
    sidx = sidx.at[:, NCHUNK - 1, 0:CH - TAIL_NEW].set(TRASH)

    g = _gather_call()(table, gidx)
    tp_ext = _tc_fused(edge_attr, g, edge_sh, jnp.asarray(_AB), fc_w1, b1,
                       w2p, b2p, jnp.asarray(_RED))
    zeros = jnp.zeros((NACC, 128), jnp.float32)
    acc = _scatter_call()(tp_ext, sidx, zeros)
    planes_out = _finalize(acc[0, :N_NODES], acc[1, :N_NODES], node_planes)

    o1 = planes_out[:, 16:].reshape(N_NODES, 3, MUL).transpose(0, 2, 1)
    return jnp.concatenate(
        [planes_out[:, 0:16], o1.reshape(N_NODES, 48)], axis=1)


# speedup vs baseline: 1.4637x; 1.0826x over previous
import functools

import jax
import jax.numpy as jnp
import numpy as np
from jax import lax
from jax.experimental import pallas as pl
from jax.experimental.pallas import tpu as pltpu
from jax.experimental.pallas import tpu_sc as plsc

N_NODES = 10000
N_EDGES = 160000
MUL = 16
IN_DIM = 64
N_EDGE_FEAT = 256
HIDDEN = 256
WEIGHT_NUMEL = 1024

NC = 2
NS = 16
NW = NC * NS
EPW = N_EDGES // NW
CH = 128
NCHUNK = (EPW + CH - 1) // CH
TAIL_OFF = EPW - CH
TAIL_NEW = EPW - (NCHUNK - 1) * CH
NACC = 10112
STRIPE = NACC // NS
TRASH = N_NODES

BE = 640
NBLK = N_EDGES // BE

_INV_SQRT3 = 1.0 / np.sqrt(3.0)
_INV_SQRT32 = 1.0 / np.sqrt(32.0)


def _gather_body(table_hbm, dst_hbm, out_hbm, idx_v, r0, r1, sem0, sem1):
    c = lax.axis_index("c")
    s = lax.axis_index("s")
    wid = c * NS + s
    base = wid * EPW
    pltpu.sync_copy(dst_hbm.at[pl.ds(base, EPW)], idx_v)

    def ioff(j):
        return jnp.minimum(j * CH, TAIL_OFF)

    pltpu.async_copy(table_hbm.at[idx_v.at[pl.ds(0, CH)]], r0, sem0)

    def body(k, carry):
        j0 = 2 * k
        j1 = 2 * k + 1
        pltpu.async_copy(table_hbm.at[idx_v.at[pl.ds(ioff(j1), CH)]], r1, sem1)
        pltpu.make_async_copy(table_hbm.at[idx_v.at[pl.ds(0, CH)]], r0, sem0).wait()
        pltpu.sync_copy(r0, out_hbm.at[pl.ds(base + ioff(j0), CH)])

        @pl.when(j0 + 2 < NCHUNK)
        def _():
            pltpu.async_copy(
                table_hbm.at[idx_v.at[pl.ds(ioff(j0 + 2), CH)]], r0, sem0)

        pltpu.make_async_copy(table_hbm.at[idx_v.at[pl.ds(0, CH)]], r1, sem1).wait()
        pltpu.sync_copy(r1, out_hbm.at[pl.ds(base + ioff(j1), CH)])
        return carry

    lax.fori_loop(0, NCHUNK // 2, body, 0)


@functools.lru_cache(maxsize=None)
def _gather_call():
    return functools.partial(
        pl.kernel,
        _gather_body,
        out_type=jax.ShapeDtypeStruct((N_EDGES, 128), jnp.float32),
        mesh=plsc.VectorSubcoreMesh(core_axis_name="c", subcore_axis_name="s"),
        scratch_types=[
            pltpu.VMEM((EPW,), jnp.int32),
            pltpu.VMEM((CH, 128), jnp.float32),
            pltpu.VMEM((CH, 128), jnp.float32),
            pltpu.SemaphoreType.DMA,
            pltpu.SemaphoreType.DMA,
        ],
    )()


def _scatter_body(tp_hbm, idx_hbm, zeros_hbm, out_hbm, idx_v, r0, r1, acc,
                  sem0, sem1):
    c = lax.axis_index("c")
    s = lax.axis_index("s")
    wid = c * NS + s
    pltpu.sync_copy(idx_hbm.at[wid], idx_v)

    @pl.when(s == 0)
    def _():
        pltpu.sync_copy(zeros_hbm, acc)

    plsc.subcore_barrier()

    def roff(j):
        return wid * EPW + jnp.minimum(j * CH, TAIL_OFF)

    pltpu.async_copy(tp_hbm.at[pl.ds(roff(0), CH)], r0, sem0)

    def body(k, carry):
        j0 = 2 * k
        j1 = 2 * k + 1
        pltpu.async_copy(tp_hbm.at[pl.ds(roff(j1), CH)], r1, sem1)
        pltpu.make_async_copy(tp_hbm.at[pl.ds(roff(j0), CH)], r0, sem0).wait()
        pltpu.sync_copy(r0, acc.at[idx_v.at[j0]], add=True)

        @pl.when(j0 + 2 < NCHUNK)
        def _():
            pltpu.async_copy(tp_hbm.at[pl.ds(roff(j0 + 2), CH)], r0, sem0)

        pltpu.make_async_copy(tp_hbm.at[pl.ds(roff(j1), CH)], r1, sem1).wait()
        pltpu.sync_copy(r1, acc.at[idx_v.at[j1]], add=True)
        return carry

    lax.fori_loop(0, NCHUNK // 2, body, 0)
    plsc.subcore_barrier()
    pltpu.sync_copy(acc.at[pl.ds(s * STRIPE, STRIPE)],
                    out_hbm.at[c].at[pl.ds(s * STRIPE, STRIPE)])


@functools.lru_cache(maxsize=None)
def _scatter_call():
    return pl.kernel(
        _scatter_body,
        out_type=jax.ShapeDtypeStruct((NC, NACC, 128), jnp.float32),
        mesh=plsc.VectorSubcoreMesh(core_axis_name="c", subcore_axis_name="s"),
        scratch_types=[
            pltpu.VMEM((NCHUNK, CH), jnp.int32),
            pltpu.VMEM((CH, 128), jnp.float32),
            pltpu.VMEM((CH, 128), jnp.float32),
            pltpu.VMEM_SHARED((NACC, 128), jnp.float32),
            pltpu.SemaphoreType.DMA,
            pltpu.SemaphoreType.DMA,
        ],
    )


def _tc_body(ea_ref, g_ref, sh_ref, ab_ref, w1_ref, b1_ref, w2_ref, b2_ref,
             red_ref, out_ref):
    x = ea_ref[...]
    h = jnp.maximum(
        jnp.dot(x, w1_ref[...], preferred_element_type=jnp.float32) + b1_ref[...], 0.0)
    tpw = jnp.dot(h, w2_ref[...], preferred_element_type=jnp.float32) + b2_ref[...]

    shab = jnp.dot(sh_ref[...], ab_ref[...], preferred_element_type=jnp.float32)

    g = g_ref[...]
    m = g * shab[:, 0:128]
    n = g * shab[:, 128:256]
    dot = m[:, 112:128] + n[:, 48:64] + n[:, 80:96]
    u0 = jnp.concatenate([m[:, 96:112], dot], 1)
    ux = m[:, 0:32]
    uy = m[:, 32:64]
    uz = m[:, 64:96]

    t0 = jnp.concatenate([u0] * 16, 1)
    tx = jnp.concatenate([ux] * 16, 1)
    ty = jnp.concatenate([uy] * 16, 1)
    tz = jnp.concatenate([uz] * 16, 1)
    w0 = tpw[:, 0:512]
    w1 = tpw[:, 512:1024]
    pall = jnp.concatenate([t0 * w0, tx * w1, ty * w1, tz * w1], 0)

    sred = jnp.dot(pall, red_ref[...], preferred_element_type=jnp.float32)

    ones_col = jnp.concatenate(
        [jnp.ones((BE, 16), jnp.float32), jnp.zeros((BE, 48), jnp.float32)], 1)
    out_ref[...] = jnp.concatenate(
        [sred[0:BE], sred[BE:2 * BE], sred[2 * BE:3 * BE], sred[3 * BE:4 * BE],
         ones_col], 1)


def _tc_fused(edge_attr, g, edge_sh, ab, w1, b1, w2p, b2p, red):
    return pl.pallas_call(
        _tc_body,
        grid=(NBLK,),
        in_specs=[
            pl.BlockSpec((BE, N_EDGE_FEAT), lambda i: (i, 0)),
            pl.BlockSpec((BE, 128), lambda i: (i, 0)),
            pl.BlockSpec((BE, 4), lambda i: (i, 0)),
            pl.BlockSpec((4, 256), lambda i: (0, 0)),
            pl.BlockSpec((N_EDGE_FEAT, HIDDEN), lambda i: (0, 0)),
            pl.BlockSpec((1, HIDDEN), lambda i: (0, 0)),
            pl.BlockSpec((HIDDEN, WEIGHT_NUMEL), lambda i: (0, 0)),
            pl.BlockSpec((1, WEIGHT_NUMEL), lambda i: (0, 0)),
            pl.BlockSpec((512, 16), lambda i: (0, 0)),
        ],
        out_specs=pl.BlockSpec((BE, 128), lambda i: (i, 0)),
        out_shape=jax.ShapeDtypeStruct((N_EDGES, 128), jnp.float32),
        compiler_params=pltpu.CompilerParams(
            dimension_semantics=("parallel",)),
    )(edge_attr, g, edge_sh, ab, w1, b1, w2p, b2p, red)


def _fin_body(a_ref, b_ref, n_ref, out_ref):
    a = a_ref[...]
    b = b_ref[...]
    tot = a[:, 0:64] + b[:, 0:64]
    cnt = a[:, 64:65] + b[:, 64:65]
    out_ref[...] = tot / jnp.maximum(cnt, 1.0) + n_ref[...]


def _finalize(acc0, acc1, node_planes):
    return pl.pallas_call(
        _fin_body,
        grid=(10,),
        in_specs=[
            pl.BlockSpec((1000, 128), lambda i: (i, 0)),
            pl.BlockSpec((1000, 128), lambda i: (i, 0)),
            pl.BlockSpec((1000, IN_DIM), lambda i: (i, 0)),
        ],
        out_specs=pl.BlockSpec((1000, IN_DIM), lambda i: (i, 0)),
        out_shape=jax.ShapeDtypeStruct((N_NODES, IN_DIM), jnp.float32),
    )(acc0, acc1, node_planes)


_Q = np.arange(512)
_PERM = np.concatenate([16 * (_Q % 32) + _Q // 32,
                        512 + 16 * (_Q % 32) + _Q // 32]).astype(np.int32)

_RED = np.zeros((512, 16), np.float32)
_RED[_Q, _Q // 32] = _INV_SQRT32

_AB = np.zeros((4, 256), np.float32)
for _g, (_s, _v) in enumerate([(1, 1.0), (0, 1.0), (2, 1.0), (0, 1.0),
                               (3, 1.0), (0, 1.0), (0, 1.0), (1, _INV_SQRT3)]):
    _AB[_s, 16 * _g:16 * (_g + 1)] = _v
_AB[2, 128 + 48:128 + 64] = _INV_SQRT3
_AB[3, 128 + 80:128 + 96] = _INV_SQRT3

_OFFS = np.minimum(np.arange(NCHUNK) * CH, TAIL_OFF)
_POS = (np.arange(NW)[:, None, None] * EPW
        + _OFFS[None, :, None]
        + np.arange(CH)[None, None, :]).astype(np.int32)


def kernel(node_attr, edge_index, edge_attr, edge_sh, fc_w1, fc_b1, fc_w2, fc_b2):
    src = edge_index[0]
    dst = edge_index[1]

    n1 = node_attr[:, 16:].reshape(N_NODES, MUL, 3)
    node_planes = jnp.concatenate(
        [node_attr[:, 0:16], n1[:, :, 0], n1[:, :, 1], n1[:, :, 2]], axis=1)
    n0 = node_attr[:, 0:16]
    table = jnp.concatenate(
        [n0, n1[:, :, 0], n0, n1[:, :, 1], n0, n1[:, :, 2], n0, n1[:, :, 0]],
        axis=1)


    w2p = fc_w2[:, _PERM]
    b2p = fc_b2[_PERM].reshape(1, WEIGHT_NUMEL)
    b1 = fc_b1.reshape(1, HIDDEN)

    sidx = src[_POS]
    sidx = sidx.at[:, NCHUNK - 1, 0:CH - TAIL_NEW].set(TRASH)

    g = _gather_call()(table, dst)
    tp_ext = _tc_fused(edge_attr, g, edge_sh, jnp.asarray(_AB), fc_w1, b1,
                       w2p, b2p, jnp.asarray(_RED))
    zeros = jnp.zeros((NACC, 128), jnp.float32)
    acc = _scatter_call()(tp_ext, sidx, zeros)
    planes_out = _finalize(acc[0, :N_NODES], acc[1, :N_NODES], node_planes)

    o1 = planes_out[:, 16:].reshape(N_NODES, 3, MUL).transpose(0, 2, 1)
    return jnp.concatenate(
        [planes_out[:, 0:16], o1.reshape(N_NODES, 48)], axis=1)


# speedup vs baseline: 1.6352x; 1.1172x over previous
import functools

import jax
import jax.numpy as jnp
import numpy as np
from jax import lax
from jax.experimental import pallas as pl
from jax.experimental.pallas import tpu as pltpu
from jax.experimental.pallas import tpu_sc as plsc

N_NODES = 10000
N_EDGES = 160000
MUL = 16
IN_DIM = 64
N_EDGE_FEAT = 256
HIDDEN = 256
WEIGHT_NUMEL = 1024

NC = 2
NS = 16
NW = NC * NS
EPW = N_EDGES // NW
CH = 128
NCHUNK = (EPW + CH - 1) // CH
TAIL_OFF = EPW - CH
TAIL_NEW = EPW - (NCHUNK - 1) * CH
NACC = 10112
STRIPE = NACC // NS
TRASH = N_NODES

BE = 1280
NBLK = N_EDGES // BE

_INV_SQRT3 = 1.0 / np.sqrt(3.0)
_INV_SQRT32 = 1.0 / np.sqrt(32.0)


def _gather_body(table_hbm, dst_hbm, out_hbm, idx_v, r0, r1, sem0, sem1):
    c = lax.axis_index("c")
    s = lax.axis_index("s")
    wid = c * NS + s
    base = wid * EPW
    pltpu.sync_copy(dst_hbm.at[pl.ds(base, EPW)], idx_v)

    def ioff(j):
        return jnp.minimum(j * CH, TAIL_OFF)

    pltpu.async_copy(table_hbm.at[idx_v.at[pl.ds(0, CH)]], r0, sem0)

    def body(k, carry):
        j0 = 2 * k
        j1 = 2 * k + 1
        pltpu.async_copy(table_hbm.at[idx_v.at[pl.ds(ioff(j1), CH)]], r1, sem1)
        pltpu.make_async_copy(table_hbm.at[idx_v.at[pl.ds(0, CH)]], r0, sem0).wait()
        pltpu.sync_copy(r0, out_hbm.at[pl.ds(base + ioff(j0), CH)])

        @pl.when(j0 + 2 < NCHUNK)
        def _():
            pltpu.async_copy(
                table_hbm.at[idx_v.at[pl.ds(ioff(j0 + 2), CH)]], r0, sem0)

        pltpu.make_async_copy(table_hbm.at[idx_v.at[pl.ds(0, CH)]], r1, sem1).wait()
        pltpu.sync_copy(r1, out_hbm.at[pl.ds(base + ioff(j1), CH)])
        return carry

    lax.fori_loop(0, NCHUNK // 2, body, 0)


@functools.lru_cache(maxsize=None)
def _gather_call():
    return functools.partial(
        pl.kernel,
        _gather_body,
        out_type=jax.ShapeDtypeStruct((N_EDGES, 128), jnp.float32),
        mesh=plsc.VectorSubcoreMesh(core_axis_name="c", subcore_axis_name="s"),
        scratch_types=[
            pltpu.VMEM((EPW,), jnp.int32),
            pltpu.VMEM((CH, 128), jnp.float32),
            pltpu.VMEM((CH, 128), jnp.float32),
            pltpu.SemaphoreType.DMA,
            pltpu.SemaphoreType.DMA,
        ],
    )()


NFULL = EPW // CH
TAIL = EPW - NFULL * CH


def _scatter_body(tp_hbm, src_hbm, zeros_hbm, out_hbm, i0, i1, i8, r0, r1, r8,
                  acc, sem0, sem1):
    c = lax.axis_index("c")
    s = lax.axis_index("s")
    wid = c * NS + s
    base = wid * EPW

    @pl.when(s == 0)
    def _():
        pltpu.sync_copy(zeros_hbm, acc)

    plsc.subcore_barrier()

    pltpu.sync_copy(src_hbm.at[pl.ds(base, CH)], i0)
    pltpu.async_copy(tp_hbm.at[pl.ds(base, CH)], r0, sem0)

    def body(k, carry):
        j0 = 2 * k
        j1 = 2 * k + 1
        pltpu.sync_copy(src_hbm.at[pl.ds(base + j1 * CH, CH)], i1)
        pltpu.async_copy(tp_hbm.at[pl.ds(base + j1 * CH, CH)], r1, sem1)
        pltpu.make_async_copy(tp_hbm.at[pl.ds(base, CH)], r0, sem0).wait()
        pltpu.sync_copy(r0, acc.at[i0], add=True)

        @pl.when(j0 + 2 < NFULL)
        def _():
            pltpu.sync_copy(src_hbm.at[pl.ds(base + (j0 + 2) * CH, CH)], i0)
            pltpu.async_copy(tp_hbm.at[pl.ds(base + (j0 + 2) * CH, CH)], r0, sem0)

        pltpu.make_async_copy(tp_hbm.at[pl.ds(base, CH)], r1, sem1).wait()
        pltpu.sync_copy(r1, acc.at[i1], add=True)
        return carry

    lax.fori_loop(0, NFULL // 2, body, 0)
    pltpu.make_async_copy(tp_hbm.at[pl.ds(base, CH)], r0, sem0).wait()
    pltpu.sync_copy(r0, acc.at[i0], add=True)
    pltpu.sync_copy(src_hbm.at[pl.ds(base + NFULL * CH, TAIL)], i8)
    pltpu.sync_copy(tp_hbm.at[pl.ds(base + NFULL * CH, TAIL)], r8)
    pltpu.sync_copy(r8, acc.at[i8], add=True)

    plsc.subcore_barrier()
    pltpu.sync_copy(acc.at[pl.ds(s * STRIPE, STRIPE)],
                    out_hbm.at[c].at[pl.ds(s * STRIPE, STRIPE)])


@functools.lru_cache(maxsize=None)
def _scatter_call():
    return pl.kernel(
        _scatter_body,
        out_type=jax.ShapeDtypeStruct((NC, NACC, 128), jnp.float32),
        mesh=plsc.VectorSubcoreMesh(core_axis_name="c", subcore_axis_name="s"),
        scratch_types=[
            pltpu.VMEM((CH,), jnp.int32),
            pltpu.VMEM((CH,), jnp.int32),
            pltpu.VMEM((TAIL,), jnp.int32),
            pltpu.VMEM((CH, 128), jnp.float32),
            pltpu.VMEM((CH, 128), jnp.float32),
            pltpu.VMEM((TAIL, 128), jnp.float32),
            pltpu.VMEM_SHARED((NACC, 128), jnp.float32),
            pltpu.SemaphoreType.DMA,
            pltpu.SemaphoreType.DMA,
        ],
    )


def _tc_body(ea_ref, g_ref, sh_ref, ab_ref, w1_ref, b1_ref, w2_ref, b2_ref,
             red_ref, out_ref):
    x = ea_ref[...]
    h = jnp.maximum(
        jnp.dot(x, w1_ref[...], preferred_element_type=jnp.float32) + b1_ref[...], 0.0)
    tpw = jnp.dot(h, w2_ref[...], preferred_element_type=jnp.float32) + b2_ref[...]

    shab = jnp.dot(sh_ref[...], ab_ref[...], preferred_element_type=jnp.float32)

    g = g_ref[...]
    m = g * shab[:, 0:128]
    n = g * shab[:, 128:256]
    dot = m[:, 112:128] + n[:, 48:64] + n[:, 80:96]
    u0 = jnp.concatenate([m[:, 96:112], dot], 1)
    ux = m[:, 0:32]
    uy = m[:, 32:64]
    uz = m[:, 64:96]

    t0 = jnp.concatenate([u0] * 16, 1)
    tx = jnp.concatenate([ux] * 16, 1)
    ty = jnp.concatenate([uy] * 16, 1)
    tz = jnp.concatenate([uz] * 16, 1)
    w0 = tpw[:, 0:512]
    w1 = tpw[:, 512:1024]
    pall = jnp.concatenate([t0 * w0, tx * w1, ty * w1, tz * w1], 0)

    sred = jnp.dot(pall, red_ref[...], preferred_element_type=jnp.float32)

    ones_col = jnp.concatenate(
        [jnp.ones((BE, 16), jnp.float32), jnp.zeros((BE, 48), jnp.float32)], 1)
    out_ref[...] = jnp.concatenate(
        [sred[0:BE], sred[BE:2 * BE], sred[2 * BE:3 * BE], sred[3 * BE:4 * BE],
         ones_col], 1)


def _tc_fused(edge_attr, g, edge_sh, ab, w1, b1, w2p, b2p, red):
    return pl.pallas_call(
        _tc_body,
        grid=(NBLK,),
        in_specs=[
            pl.BlockSpec((BE, N_EDGE_FEAT), lambda i: (i, 0)),
            pl.BlockSpec((BE, 128), lambda i: (i, 0)),
            pl.BlockSpec((BE, 4), lambda i: (i, 0)),
            pl.BlockSpec((4, 256), lambda i: (0, 0)),
            pl.BlockSpec((N_EDGE_FEAT, HIDDEN), lambda i: (0, 0)),
            pl.BlockSpec((1, HIDDEN), lambda i: (0, 0)),
            pl.BlockSpec((HIDDEN, WEIGHT_NUMEL), lambda i: (0, 0)),
            pl.BlockSpec((1, WEIGHT_NUMEL), lambda i: (0, 0)),
            pl.BlockSpec((512, 16), lambda i: (0, 0)),
        ],
        out_specs=pl.BlockSpec((BE, 128), lambda i: (i, 0)),
        out_shape=jax.ShapeDtypeStruct((N_EDGES, 128), jnp.float32),
        compiler_params=pltpu.CompilerParams(
            dimension_semantics=("parallel",)),
    )(edge_attr, g, edge_sh, ab, w1, b1, w2p, b2p, red)


def _fin_body(a_ref, b_ref, n_ref, out_ref):
    a = a_ref[...]
    b = b_ref[...]
    tot = a[:, 0:64] + b[:, 0:64]
    cnt = a[:, 64:65] + b[:, 64:65]
    out_ref[...] = tot / jnp.maximum(cnt, 1.0) + n_ref[...]


def _fin_body2(acc_ref, n_ref, out_ref):
    a = acc_ref[0]
    b = acc_ref[1]
    tot = a[:, 0:64] + b[:, 0:64]
    cnt = a[:, 64:65] + b[:, 64:65]
    out_ref[...] = tot / jnp.maximum(cnt, 1.0) + n_ref[...]


def _finalize(acc, node_planes):
    return pl.pallas_call(
        _fin_body2,
        grid=(10,),
        in_specs=[
            pl.BlockSpec((2, 1000, 128), lambda i: (0, i, 0)),
            pl.BlockSpec((1000, IN_DIM), lambda i: (i, 0)),
        ],
        out_specs=pl.BlockSpec((1000, IN_DIM), lambda i: (i, 0)),
        out_shape=jax.ShapeDtypeStruct((N_NODES, IN_DIM), jnp.float32),
    )(acc, node_planes)


_Q = np.arange(512)
_PERM = np.concatenate([16 * (_Q % 32) + _Q // 32,
                        512 + 16 * (_Q % 32) + _Q // 32]).astype(np.int32)

_RED = np.zeros((512, 16), np.float32)
_RED[_Q, _Q // 32] = _INV_SQRT32

_AB = np.zeros((4, 256), np.float32)
for _g, (_s, _v) in enumerate([(1, 1.0), (0, 1.0), (2, 1.0), (0, 1.0),
                               (3, 1.0), (0, 1.0), (0, 1.0), (1, _INV_SQRT3)]):
    _AB[_s, 16 * _g:16 * (_g + 1)] = _v
_AB[2, 128 + 48:128 + 64] = _INV_SQRT3
_AB[3, 128 + 80:128 + 96] = _INV_SQRT3

_OFFS = np.minimum(np.arange(NCHUNK) * CH, TAIL_OFF)
_POS = (np.arange(NW)[:, None, None] * EPW
        + _OFFS[None, :, None]
        + np.arange(CH)[None, None, :]).astype(np.int32)


def kernel(node_attr, edge_index, edge_attr, edge_sh, fc_w1, fc_b1, fc_w2, fc_b2):
    src = edge_index[0]
    dst = edge_index[1]

    n1 = node_attr[:, 16:].reshape(N_NODES, MUL, 3)
    node_planes = jnp.concatenate(
        [node_attr[:, 0:16], n1[:, :, 0], n1[:, :, 1], n1[:, :, 2]], axis=1)
    n0 = node_attr[:, 0:16]
    table = jnp.concatenate(
        [n0, n1[:, :, 0], n0, n1[:, :, 1], n0, n1[:, :, 2], n0, n1[:, :, 0]],
        axis=1)


    w2p = fc_w2[:, _PERM]
    b2p = fc_b2[_PERM].reshape(1, WEIGHT_NUMEL)
    b1 = fc_b1.reshape(1, HIDDEN)

    g = _gather_call()(table, dst)
    tp_ext = _tc_fused(edge_attr, g, edge_sh, jnp.asarray(_AB), fc_w1, b1,
                       w2p, b2p, jnp.asarray(_RED))
    zeros = jnp.zeros((NACC, 128), jnp.float32)
    acc = _scatter_call()(tp_ext, src, zeros)
    planes_out = _finalize(acc, node_planes)

    o1 = planes_out[:, 16:].reshape(N_NODES, 3, MUL).transpose(0, 2, 1)
    return jnp.concatenate(
        [planes_out[:, 0:16], o1.reshape(N_NODES, 48)], axis=1)


# speedup vs baseline: 1.6929x; 1.0353x over previous
import functools

import jax
import jax.numpy as jnp
import numpy as np
from jax import lax
from jax.experimental import pallas as pl
from jax.experimental.pallas import tpu as pltpu
from jax.experimental.pallas import tpu_sc as plsc

N_NODES = 10000
N_EDGES = 160000
MUL = 16
IN_DIM = 64
N_EDGE_FEAT = 256
HIDDEN = 256
WEIGHT_NUMEL = 1024

NC = 2
NS = 16
NW = NC * NS
EPW = N_EDGES // NW
CH = 128
NCHUNK = (EPW + CH - 1) // CH
TAIL_OFF = EPW - CH
TAIL_NEW = EPW - (NCHUNK - 1) * CH
NACC = 10112
STRIPE = NACC // NS
TRASH = N_NODES

BE = 2000
NBLK = N_EDGES // BE

_INV_SQRT3 = 1.0 / np.sqrt(3.0)
_INV_SQRT32 = 1.0 / np.sqrt(32.0)


def _gather_body(table_hbm, dst_hbm, out_hbm, idx_v, r0, r1, sem0, sem1):
    c = lax.axis_index("c")
    s = lax.axis_index("s")
    wid = c * NS + s
    base = wid * EPW
    pltpu.sync_copy(dst_hbm.at[pl.ds(base, EPW)], idx_v)

    def ioff(j):
        return jnp.minimum(j * CH, TAIL_OFF)

    pltpu.async_copy(table_hbm.at[idx_v.at[pl.ds(0, CH)]], r0, sem0)

    def body(k, carry):
        j0 = 2 * k
        j1 = 2 * k + 1
        pltpu.async_copy(table_hbm.at[idx_v.at[pl.ds(ioff(j1), CH)]], r1, sem1)
        pltpu.make_async_copy(table_hbm.at[idx_v.at[pl.ds(0, CH)]], r0, sem0).wait()
        pltpu.sync_copy(r0, out_hbm.at[pl.ds(base + ioff(j0), CH)])

        @pl.when(j0 + 2 < NCHUNK)
        def _():
            pltpu.async_copy(
                table_hbm.at[idx_v.at[pl.ds(ioff(j0 + 2), CH)]], r0, sem0)

        pltpu.make_async_copy(table_hbm.at[idx_v.at[pl.ds(0, CH)]], r1, sem1).wait()
        pltpu.sync_copy(r1, out_hbm.at[pl.ds(base + ioff(j1), CH)])
        return carry

    lax.fori_loop(0, NCHUNK // 2, body, 0)


@functools.lru_cache(maxsize=None)
def _gather_call():
    return functools.partial(
        pl.kernel,
        _gather_body,
        out_type=jax.ShapeDtypeStruct((N_EDGES, 128), jnp.float32),
        mesh=plsc.VectorSubcoreMesh(core_axis_name="c", subcore_axis_name="s"),
        scratch_types=[
            pltpu.VMEM((EPW,), jnp.int32),
            pltpu.VMEM((CH, 128), jnp.float32),
            pltpu.VMEM((CH, 128), jnp.float32),
            pltpu.SemaphoreType.DMA,
            pltpu.SemaphoreType.DMA,
        ],
    )()


NFULL = EPW // CH
TAIL = EPW - NFULL * CH


def _scatter_body(tp_hbm, src_hbm, zeros_hbm, out_hbm, i0, i1, i8, r0, r1, r8,
                  acc, sem0, sem1):
    c = lax.axis_index("c")
    s = lax.axis_index("s")
    wid = c * NS + s
    base = wid * EPW

    @pl.when(s == 0)
    def _():
        pltpu.sync_copy(zeros_hbm, acc)

    plsc.subcore_barrier()

    pltpu.sync_copy(src_hbm.at[pl.ds(base, CH)], i0)
    pltpu.async_copy(tp_hbm.at[pl.ds(base, CH)], r0, sem0)

    def body(k, carry):
        j0 = 2 * k
        j1 = 2 * k + 1
        pltpu.sync_copy(src_hbm.at[pl.ds(base + j1 * CH, CH)], i1)
        pltpu.async_copy(tp_hbm.at[pl.ds(base + j1 * CH, CH)], r1, sem1)
        pltpu.make_async_copy(tp_hbm.at[pl.ds(base, CH)], r0, sem0).wait()
        pltpu.sync_copy(r0, acc.at[i0], add=True)

        @pl.when(j0 + 2 < NFULL)
        def _():
            pltpu.sync_copy(src_hbm.at[pl.ds(base + (j0 + 2) * CH, CH)], i0)
            pltpu.async_copy(tp_hbm.at[pl.ds(base + (j0 + 2) * CH, CH)], r0, sem0)

        pltpu.make_async_copy(tp_hbm.at[pl.ds(base, CH)], r1, sem1).wait()
        pltpu.sync_copy(r1, acc.at[i1], add=True)
        return carry

    lax.fori_loop(0, NFULL // 2, body, 0)
    pltpu.make_async_copy(tp_hbm.at[pl.ds(base, CH)], r0, sem0).wait()
    pltpu.sync_copy(r0, acc.at[i0], add=True)
    pltpu.sync_copy(src_hbm.at[pl.ds(base + NFULL * CH, TAIL)], i8)
    pltpu.sync_copy(tp_hbm.at[pl.ds(base + NFULL * CH, TAIL)], r8)
    pltpu.sync_copy(r8, acc.at[i8], add=True)

    plsc.subcore_barrier()
    pltpu.sync_copy(acc.at[pl.ds(s * STRIPE, STRIPE)],
                    out_hbm.at[c].at[pl.ds(s * STRIPE, STRIPE)])


@functools.lru_cache(maxsize=None)
def _scatter_call():
    return pl.kernel(
        _scatter_body,
        out_type=jax.ShapeDtypeStruct((NC, NACC, 128), jnp.float32),
        mesh=plsc.VectorSubcoreMesh(core_axis_name="c", subcore_axis_name="s"),
        scratch_types=[
            pltpu.VMEM((CH,), jnp.int32),
            pltpu.VMEM((CH,), jnp.int32),
            pltpu.VMEM((TAIL,), jnp.int32),
            pltpu.VMEM((CH, 128), jnp.float32),
            pltpu.VMEM((CH, 128), jnp.float32),
            pltpu.VMEM((TAIL, 128), jnp.float32),
            pltpu.VMEM_SHARED((NACC, 128), jnp.float32),
            pltpu.SemaphoreType.DMA,
            pltpu.SemaphoreType.DMA,
        ],
    )


def _tc_body(ea_ref, g_ref, sh_ref, ab_ref, w1_ref, b1_ref, w2_ref, b2_ref,
             red_ref, out_ref):
    x = ea_ref[...]
    h = jnp.maximum(
        jnp.dot(x, w1_ref[...], preferred_element_type=jnp.float32) + b1_ref[...], 0.0)
    tpw = jnp.dot(h, w2_ref[...], preferred_element_type=jnp.float32) + b2_ref[...]

    shab = jnp.dot(sh_ref[...], ab_ref[...], preferred_element_type=jnp.float32)

    g = g_ref[...]
    m = g * shab[:, 0:128]
    n = g * shab[:, 128:256]
    dot = m[:, 112:128] + n[:, 48:64] + n[:, 80:96]
    u0 = jnp.concatenate([m[:, 96:112], dot], 1)
    ux = m[:, 0:32]
    uy = m[:, 32:64]
    uz = m[:, 64:96]

    t0 = jnp.concatenate([u0] * 16, 1)
    tx = jnp.concatenate([ux] * 16, 1)
    ty = jnp.concatenate([uy] * 16, 1)
    tz = jnp.concatenate([uz] * 16, 1)
    w0 = tpw[:, 0:512]
    w1 = tpw[:, 512:1024]
    pall = jnp.concatenate([t0 * w0, tx * w1, ty * w1, tz * w1], 0)

    sred = jnp.dot(pall, red_ref[...], preferred_element_type=jnp.float32)

    ones_col = jnp.concatenate(
        [jnp.ones((BE, 16), jnp.float32), jnp.zeros((BE, 48), jnp.float32)], 1)
    out_ref[...] = jnp.concatenate(
        [sred[0:BE], sred[BE:2 * BE], sred[2 * BE:3 * BE], sred[3 * BE:4 * BE],
         ones_col], 1)


def _tc_fused(edge_attr, g, edge_sh, ab, w1, b1, w2p, b2p, red):
    return pl.pallas_call(
        _tc_body,
        grid=(NBLK,),
        in_specs=[
            pl.BlockSpec((BE, N_EDGE_FEAT), lambda i: (i, 0)),
            pl.BlockSpec((BE, 128), lambda i: (i, 0)),
            pl.BlockSpec((BE, 4), lambda i: (i, 0)),
            pl.BlockSpec((4, 256), lambda i: (0, 0)),
            pl.BlockSpec((N_EDGE_FEAT, HIDDEN), lambda i: (0, 0)),
            pl.BlockSpec((1, HIDDEN), lambda i: (0, 0)),
            pl.BlockSpec((HIDDEN, WEIGHT_NUMEL), lambda i: (0, 0)),
            pl.BlockSpec((1, WEIGHT_NUMEL), lambda i: (0, 0)),
            pl.BlockSpec((512, 16), lambda i: (0, 0)),
        ],
        out_specs=pl.BlockSpec((BE, 128), lambda i: (i, 0)),
        out_shape=jax.ShapeDtypeStruct((N_EDGES, 128), jnp.float32),
        compiler_params=pltpu.CompilerParams(
            dimension_semantics=("parallel",)),
    )(edge_attr, g, edge_sh, ab, w1, b1, w2p, b2p, red)


def _fin_body(a_ref, b_ref, n_ref, out_ref):
    a = a_ref[...]
    b = b_ref[...]
    tot = a[:, 0:64] + b[:, 0:64]
    cnt = a[:, 64:65] + b[:, 64:65]
    out_ref[...] = tot / jnp.maximum(cnt, 1.0) + n_ref[...]


def _fin_body2(acc_ref, n_ref, out_ref):
    a = acc_ref[0]
    b = acc_ref[1]
    tot = a[:, 0:64] + b[:, 0:64]
    cnt = a[:, 64:65] + b[:, 64:65]
    out_ref[...] = tot / jnp.maximum(cnt, 1.0) + n_ref[...]


def _finalize(acc, node_planes):
    return pl.pallas_call(
        _fin_body2,
        grid=(10,),
        in_specs=[
            pl.BlockSpec((2, 1000, 128), lambda i: (0, i, 0)),
            pl.BlockSpec((1000, IN_DIM), lambda i: (i, 0)),
        ],
        out_specs=pl.BlockSpec((1000, IN_DIM), lambda i: (i, 0)),
        out_shape=jax.ShapeDtypeStruct((N_NODES, IN_DIM), jnp.float32),
    )(acc, node_planes)


_Q = np.arange(512)
_PERM = np.concatenate([16 * (_Q % 32) + _Q // 32,
                        512 + 16 * (_Q % 32) + _Q // 32]).astype(np.int32)

_RED = np.zeros((512, 16), np.float32)
_RED[_Q, _Q // 32] = _INV_SQRT32

_AB = np.zeros((4, 256), np.float32)
for _g, (_s, _v) in enumerate([(1, 1.0), (0, 1.0), (2, 1.0), (0, 1.0),
                               (3, 1.0), (0, 1.0), (0, 1.0), (1, _INV_SQRT3)]):
    _AB[_s, 16 * _g:16 * (_g + 1)] = _v
_AB[2, 128 + 48:128 + 64] = _INV_SQRT3
_AB[3, 128 + 80:128 + 96] = _INV_SQRT3

_OFFS = np.minimum(np.arange(NCHUNK) * CH, TAIL_OFF)
_POS = (np.arange(NW)[:, None, None] * EPW
        + _OFFS[None, :, None]
        + np.arange(CH)[None, None, :]).astype(np.int32)


def kernel(node_attr, edge_index, edge_attr, edge_sh, fc_w1, fc_b1, fc_w2, fc_b2):
    src = edge_index[0]
    dst = edge_index[1]

    n1 = node_attr[:, 16:].reshape(N_NODES, MUL, 3)
    node_planes = jnp.concatenate(
        [node_attr[:, 0:16], n1[:, :, 0], n1[:, :, 1], n1[:, :, 2]], axis=1)
    n0 = node_attr[:, 0:16]
    table = jnp.concatenate(
        [n0, n1[:, :, 0], n0, n1[:, :, 1], n0, n1[:, :, 2], n0, n1[:, :, 0]],
        axis=1)


    w2p = fc_w2[:, _PERM]
    b2p = fc_b2[_PERM].reshape(1, WEIGHT_NUMEL)
    b1 = fc_b1.reshape(1, HIDDEN)

    g = _gather_call()(table, dst)
    tp_ext = _tc_fused(edge_attr, g, edge_sh, jnp.asarray(_AB), fc_w1, b1,
                       w2p, b2p, jnp.asarray(_RED))
    zeros = jnp.zeros((NACC, 128), jnp.float32)
    acc = _scatter_call()(tp_ext, src, zeros)
    planes_out = _finalize(acc, node_planes)

    o1 = planes_out[:, 16:].reshape(N_NODES, 3, MUL).transpose(0, 2, 1)
    return jnp.concatenate(
        [planes_out[:, 0:16], o1.reshape(N_NODES, 48)], axis=1)


# speedup vs baseline: 1.7135x; 1.0122x over previous
import functools

import jax
import jax.numpy as jnp
import numpy as np
from jax import lax
from jax.experimental import pallas as pl
from jax.experimental.pallas import tpu as pltpu
from jax.experimental.pallas import tpu_sc as plsc

N_NODES = 10000
N_EDGES = 160000
MUL = 16
IN_DIM = 64
N_EDGE_FEAT = 256
HIDDEN = 256
WEIGHT_NUMEL = 1024

NC = 2
NS = 16
NW = NC * NS
EPW = N_EDGES // NW
CH = 128
NCHUNK = (EPW + CH - 1) // CH
TAIL_OFF = EPW - CH
TAIL_NEW = EPW - (NCHUNK - 1) * CH
NACC = 10112
STRIPE = NACC // NS
TRASH = N_NODES

BE = 3200
NBLK = N_EDGES // BE

_INV_SQRT3 = 1.0 / np.sqrt(3.0)
_INV_SQRT32 = 1.0 / np.sqrt(32.0)


def _gather_body(table_hbm, dst_hbm, out_hbm, idx_v, r0, r1, sem0, sem1):
    c = lax.axis_index("c")
    s = lax.axis_index("s")
    wid = c * NS + s
    base = wid * EPW
    pltpu.sync_copy(dst_hbm.at[pl.ds(base, EPW)], idx_v)

    def ioff(j):
        return jnp.minimum(j * CH, TAIL_OFF)

    pltpu.async_copy(table_hbm.at[idx_v.at[pl.ds(0, CH)]], r0, sem0)

    def body(k, carry):
        j0 = 2 * k
        j1 = 2 * k + 1
        pltpu.async_copy(table_hbm.at[idx_v.at[pl.ds(ioff(j1), CH)]], r1, sem1)
        pltpu.make_async_copy(table_hbm.at[idx_v.at[pl.ds(0, CH)]], r0, sem0).wait()
        pltpu.sync_copy(r0, out_hbm.at[pl.ds(base + ioff(j0), CH)])

        @pl.when(j0 + 2 < NCHUNK)
        def _():
            pltpu.async_copy(
                table_hbm.at[idx_v.at[pl.ds(ioff(j0 + 2), CH)]], r0, sem0)

        pltpu.make_async_copy(table_hbm.at[idx_v.at[pl.ds(0, CH)]], r1, sem1).wait()
        pltpu.sync_copy(r1, out_hbm.at[pl.ds(base + ioff(j1), CH)])
        return carry

    lax.fori_loop(0, NCHUNK // 2, body, 0)


@functools.lru_cache(maxsize=None)
def _gather_call():
    return functools.partial(
        pl.kernel,
        _gather_body,
        out_type=jax.ShapeDtypeStruct((N_EDGES, 128), jnp.float32),
        mesh=plsc.VectorSubcoreMesh(core_axis_name="c", subcore_axis_name="s"),
        scratch_types=[
            pltpu.VMEM((EPW,), jnp.int32),
            pltpu.VMEM((CH, 128), jnp.float32),
            pltpu.VMEM((CH, 128), jnp.float32),
            pltpu.SemaphoreType.DMA,
            pltpu.SemaphoreType.DMA,
        ],
    )()


NFULL = EPW // CH
TAIL = EPW - NFULL * CH


def _scatter_body(tp_hbm, src_hbm, zeros_hbm, out_hbm, i0, i1, i8, r0, r1, r8,
                  acc, sem0, sem1):
    c = lax.axis_index("c")
    s = lax.axis_index("s")
    wid = c * NS + s
    base = wid * EPW

    @pl.when(s == 0)
    def _():
        pltpu.sync_copy(zeros_hbm, acc)

    plsc.subcore_barrier()

    pltpu.sync_copy(src_hbm.at[pl.ds(base, CH)], i0)
    pltpu.async_copy(tp_hbm.at[pl.ds(base, CH)], r0, sem0)

    def body(k, carry):
        j0 = 2 * k
        j1 = 2 * k + 1
        pltpu.sync_copy(src_hbm.at[pl.ds(base + j1 * CH, CH)], i1)
        pltpu.async_copy(tp_hbm.at[pl.ds(base + j1 * CH, CH)], r1, sem1)
        pltpu.make_async_copy(tp_hbm.at[pl.ds(base, CH)], r0, sem0).wait()
        pltpu.sync_copy(r0, acc.at[i0], add=True)

        @pl.when(j0 + 2 < NFULL)
        def _():
            pltpu.sync_copy(src_hbm.at[pl.ds(base + (j0 + 2) * CH, CH)], i0)
            pltpu.async_copy(tp_hbm.at[pl.ds(base + (j0 + 2) * CH, CH)], r0, sem0)

        pltpu.make_async_copy(tp_hbm.at[pl.ds(base, CH)], r1, sem1).wait()
        pltpu.sync_copy(r1, acc.at[i1], add=True)
        return carry

    lax.fori_loop(0, NFULL // 2, body, 0)
    pltpu.make_async_copy(tp_hbm.at[pl.ds(base, CH)], r0, sem0).wait()
    pltpu.sync_copy(r0, acc.at[i0], add=True)
    pltpu.sync_copy(src_hbm.at[pl.ds(base + NFULL * CH, TAIL)], i8)
    pltpu.sync_copy(tp_hbm.at[pl.ds(base + NFULL * CH, TAIL)], r8)
    pltpu.sync_copy(r8, acc.at[i8], add=True)

    plsc.subcore_barrier()
    pltpu.sync_copy(acc.at[pl.ds(s * STRIPE, STRIPE)],
                    out_hbm.at[c].at[pl.ds(s * STRIPE, STRIPE)])


@functools.lru_cache(maxsize=None)
def _scatter_call():
    return pl.kernel(
        _scatter_body,
        out_type=jax.ShapeDtypeStruct((NC, NACC, 128), jnp.float32),
        mesh=plsc.VectorSubcoreMesh(core_axis_name="c", subcore_axis_name="s"),
        scratch_types=[
            pltpu.VMEM((CH,), jnp.int32),
            pltpu.VMEM((CH,), jnp.int32),
            pltpu.VMEM((TAIL,), jnp.int32),
            pltpu.VMEM((CH, 128), jnp.float32),
            pltpu.VMEM((CH, 128), jnp.float32),
            pltpu.VMEM((TAIL, 128), jnp.float32),
            pltpu.VMEM_SHARED((NACC, 128), jnp.float32),
            pltpu.SemaphoreType.DMA,
            pltpu.SemaphoreType.DMA,
        ],
    )


def _tc_body(ea_ref, g_ref, sh_ref, ab_ref, w1_ref, b1_ref, w2_ref, b2_ref,
             red_ref, out_ref):
    x = ea_ref[...]
    h = jnp.maximum(
        jnp.dot(x, w1_ref[...], preferred_element_type=jnp.float32) + b1_ref[...], 0.0)
    tpw = jnp.dot(h, w2_ref[...], preferred_element_type=jnp.float32) + b2_ref[...]

    shab = jnp.dot(sh_ref[...], ab_ref[...], preferred_element_type=jnp.float32)

    g = g_ref[...]
    m = g * shab[:, 0:128]
    n = g * shab[:, 128:256]
    dot = m[:, 112:128] + n[:, 48:64] + n[:, 80:96]
    u0 = jnp.concatenate([m[:, 96:112], dot], 1)
    ux = m[:, 0:32]
    uy = m[:, 32:64]
    uz = m[:, 64:96]

    t0 = jnp.concatenate([u0] * 16, 1)
    tx = jnp.concatenate([ux] * 16, 1)
    ty = jnp.concatenate([uy] * 16, 1)
    tz = jnp.concatenate([uz] * 16, 1)
    w0 = tpw[:, 0:512]
    w1 = tpw[:, 512:1024]
    pall = jnp.concatenate([t0 * w0, tx * w1, ty * w1, tz * w1], 0)

    sred = jnp.dot(pall, red_ref[...], preferred_element_type=jnp.float32)

    ones_col = jnp.concatenate(
        [jnp.ones((BE, 16), jnp.float32), jnp.zeros((BE, 48), jnp.float32)], 1)
    out_ref[...] = jnp.concatenate(
        [sred[0:BE], sred[BE:2 * BE], sred[2 * BE:3 * BE], sred[3 * BE:4 * BE],
         ones_col], 1)


def _tc_fused(edge_attr, g, edge_sh, ab, w1, b1, w2p, b2p, red):
    return pl.pallas_call(
        _tc_body,
        grid=(NBLK,),
        in_specs=[
            pl.BlockSpec((BE, N_EDGE_FEAT), lambda i: (i, 0)),
            pl.BlockSpec((BE, 128), lambda i: (i, 0)),
            pl.BlockSpec((BE, 4), lambda i: (i, 0)),
            pl.BlockSpec((4, 256), lambda i: (0, 0)),
            pl.BlockSpec((N_EDGE_FEAT, HIDDEN), lambda i: (0, 0)),
            pl.BlockSpec((1, HIDDEN), lambda i: (0, 0)),
            pl.BlockSpec((HIDDEN, WEIGHT_NUMEL), lambda i: (0, 0)),
            pl.BlockSpec((1, WEIGHT_NUMEL), lambda i: (0, 0)),
            pl.BlockSpec((512, 16), lambda i: (0, 0)),
        ],
        out_specs=pl.BlockSpec((BE, 128), lambda i: (i, 0)),
        out_shape=jax.ShapeDtypeStruct((N_EDGES, 128), jnp.float32),
        compiler_params=pltpu.CompilerParams(
            dimension_semantics=("parallel",)),
    )(edge_attr, g, edge_sh, ab, w1, b1, w2p, b2p, red)


def _fin_body(a_ref, b_ref, n_ref, out_ref):
    a = a_ref[...]
    b = b_ref[...]
    tot = a[:, 0:64] + b[:, 0:64]
    cnt = a[:, 64:65] + b[:, 64:65]
    out_ref[...] = tot / jnp.maximum(cnt, 1.0) + n_ref[...]


def _fin_body2(acc_ref, n_ref, out_ref):
    a = acc_ref[0]
    b = acc_ref[1]
    tot = a[:, 0:64] + b[:, 0:64]
    cnt = a[:, 64:65] + b[:, 64:65]
    out_ref[...] = tot / jnp.maximum(cnt, 1.0) + n_ref[...]


def _finalize(acc, node_planes):
    return pl.pallas_call(
        _fin_body2,
        grid=(10,),
        in_specs=[
            pl.BlockSpec((2, 1000, 128), lambda i: (0, i, 0)),
            pl.BlockSpec((1000, IN_DIM), lambda i: (i, 0)),
        ],
        out_specs=pl.BlockSpec((1000, IN_DIM), lambda i: (i, 0)),
        out_shape=jax.ShapeDtypeStruct((N_NODES, IN_DIM), jnp.float32),
    )(acc, node_planes)


_Q = np.arange(512)
_PERM = np.concatenate([16 * (_Q % 32) + _Q // 32,
                        512 + 16 * (_Q % 32) + _Q // 32]).astype(np.int32)

_RED = np.zeros((512, 16), np.float32)
_RED[_Q, _Q // 32] = _INV_SQRT32

_AB = np.zeros((4, 256), np.float32)
for _g, (_s, _v) in enumerate([(1, 1.0), (0, 1.0), (2, 1.0), (0, 1.0),
                               (3, 1.0), (0, 1.0), (0, 1.0), (1, _INV_SQRT3)]):
    _AB[_s, 16 * _g:16 * (_g + 1)] = _v
_AB[2, 128 + 48:128 + 64] = _INV_SQRT3
_AB[3, 128 + 80:128 + 96] = _INV_SQRT3

_OFFS = np.minimum(np.arange(NCHUNK) * CH, TAIL_OFF)
_POS = (np.arange(NW)[:, None, None] * EPW
        + _OFFS[None, :, None]
        + np.arange(CH)[None, None, :]).astype(np.int32)


def kernel(node_attr, edge_index, edge_attr, edge_sh, fc_w1, fc_b1, fc_w2, fc_b2):
    src = edge_index[0]
    dst = edge_index[1]

    n1 = node_attr[:, 16:].reshape(N_NODES, MUL, 3)
    node_planes = jnp.concatenate(
        [node_attr[:, 0:16], n1[:, :, 0], n1[:, :, 1], n1[:, :, 2]], axis=1)
    n0 = node_attr[:, 0:16]
    table = jnp.concatenate(
        [n0, n1[:, :, 0], n0, n1[:, :, 1], n0, n1[:, :, 2], n0, n1[:, :, 0]],
        axis=1)


    w2p = fc_w2[:, _PERM]
    b2p = fc_b2[_PERM].reshape(1, WEIGHT_NUMEL)
    b1 = fc_b1.reshape(1, HIDDEN)

    g = _gather_call()(table, dst)
    tp_ext = _tc_fused(edge_attr, g, edge_sh, jnp.asarray(_AB), fc_w1, b1,
                       w2p, b2p, jnp.asarray(_RED))
    zeros = jnp.zeros((NACC, 128), jnp.float32)
    acc = _scatter_call()(tp_ext, src, zeros)
    planes_out = _finalize(acc, node_planes)

    o1 = planes_out[:, 16:].reshape(N_NODES, 3, MUL).transpose(0, 2, 1)
    return jnp.concatenate(
        [planes_out[:, 0:16], o1.reshape(N_NODES, 48)], axis=1)


# speedup vs baseline: 1.7292x; 1.0091x over previous
import functools

import jax
import jax.numpy as jnp
import numpy as np
from jax import lax
from jax.experimental import pallas as pl
from jax.experimental.pallas import tpu as pltpu
from jax.experimental.pallas import tpu_sc as plsc

N_NODES = 10000
N_EDGES = 160000
MUL = 16
IN_DIM = 64
N_EDGE_FEAT = 256
HIDDEN = 256
WEIGHT_NUMEL = 1024

NC = 2
NS = 16
NW = NC * NS
EPW = N_EDGES // NW
CH = 128
NCHUNK = (EPW + CH - 1) // CH
TAIL_OFF = EPW - CH
TAIL_NEW = EPW - (NCHUNK - 1) * CH
NACC = 10112
STRIPE = NACC // NS
TRASH = N_NODES

BE = 4000
NBLK = N_EDGES // BE

_INV_SQRT3 = 1.0 / np.sqrt(3.0)
_INV_SQRT32 = 1.0 / np.sqrt(32.0)


def _gather_body(table_hbm, dst_hbm, out_hbm, idx_v, r0, r1, sem0, sem1):
    c = lax.axis_index("c")
    s = lax.axis_index("s")
    wid = c * NS + s
    base = wid * EPW
    pltpu.sync_copy(dst_hbm.at[pl.ds(base, EPW)], idx_v)

    def ioff(j):
        return jnp.minimum(j * CH, TAIL_OFF)

    pltpu.async_copy(table_hbm.at[idx_v.at[pl.ds(0, CH)]], r0, sem0)

    def body(k, carry):
        j0 = 2 * k
        j1 = 2 * k + 1
        pltpu.async_copy(table_hbm.at[idx_v.at[pl.ds(ioff(j1), CH)]], r1, sem1)
        pltpu.make_async_copy(table_hbm.at[idx_v.at[pl.ds(0, CH)]], r0, sem0).wait()
        pltpu.sync_copy(r0, out_hbm.at[pl.ds(base + ioff(j0), CH)])

        @pl.when(j0 + 2 < NCHUNK)
        def _():
            pltpu.async_copy(
                table_hbm.at[idx_v.at[pl.ds(ioff(j0 + 2), CH)]], r0, sem0)

        pltpu.make_async_copy(table_hbm.at[idx_v.at[pl.ds(0, CH)]], r1, sem1).wait()
        pltpu.sync_copy(r1, out_hbm.at[pl.ds(base + ioff(j1), CH)])
        return carry

    lax.fori_loop(0, NCHUNK // 2, body, 0)


@functools.lru_cache(maxsize=None)
def _gather_call():
    return functools.partial(
        pl.kernel,
        _gather_body,
        out_type=jax.ShapeDtypeStruct((N_EDGES, 128), jnp.float32),
        mesh=plsc.VectorSubcoreMesh(core_axis_name="c", subcore_axis_name="s"),
        scratch_types=[
            pltpu.VMEM((EPW,), jnp.int32),
            pltpu.VMEM((CH, 128), jnp.float32),
            pltpu.VMEM((CH, 128), jnp.float32),
            pltpu.SemaphoreType.DMA,
            pltpu.SemaphoreType.DMA,
        ],
    )()


NFULL = EPW // CH
TAIL = EPW - NFULL * CH


def _scatter_body(tp_hbm, src_hbm, zeros_hbm, out_hbm, i0, i1, i8, r0, r1, r8,
                  acc, sem0, sem1):
    c = lax.axis_index("c")
    s = lax.axis_index("s")
    wid = c * NS + s
    base = wid * EPW

    @pl.when(s == 0)
    def _():
        pltpu.sync_copy(zeros_hbm, acc)

    plsc.subcore_barrier()

    pltpu.sync_copy(src_hbm.at[pl.ds(base, CH)], i0)
    pltpu.async_copy(tp_hbm.at[pl.ds(base, CH)], r0, sem0)

    def body(k, carry):
        j0 = 2 * k
        j1 = 2 * k + 1
        pltpu.sync_copy(src_hbm.at[pl.ds(base + j1 * CH, CH)], i1)
        pltpu.async_copy(tp_hbm.at[pl.ds(base + j1 * CH, CH)], r1, sem1)
        pltpu.make_async_copy(tp_hbm.at[pl.ds(base, CH)], r0, sem0).wait()
        pltpu.sync_copy(r0, acc.at[i0], add=True)

        @pl.when(j0 + 2 < NFULL)
        def _():
            pltpu.sync_copy(src_hbm.at[pl.ds(base + (j0 + 2) * CH, CH)], i0)
            pltpu.async_copy(tp_hbm.at[pl.ds(base + (j0 + 2) * CH, CH)], r0, sem0)

        pltpu.make_async_copy(tp_hbm.at[pl.ds(base, CH)], r1, sem1).wait()
        pltpu.sync_copy(r1, acc.at[i1], add=True)
        return carry

    lax.fori_loop(0, NFULL // 2, body, 0)
    pltpu.make_async_copy(tp_hbm.at[pl.ds(base, CH)], r0, sem0).wait()
    pltpu.sync_copy(r0, acc.at[i0], add=True)
    pltpu.sync_copy(src_hbm.at[pl.ds(base + NFULL * CH, TAIL)], i8)
    pltpu.sync_copy(tp_hbm.at[pl.ds(base + NFULL * CH, TAIL)], r8)
    pltpu.sync_copy(r8, acc.at[i8], add=True)

    plsc.subcore_barrier()
    pltpu.sync_copy(acc.at[pl.ds(s * STRIPE, STRIPE)],
                    out_hbm.at[c].at[pl.ds(s * STRIPE, STRIPE)])


@functools.lru_cache(maxsize=None)
def _scatter_call():
    return pl.kernel(
        _scatter_body,
        out_type=jax.ShapeDtypeStruct((NC, NACC, 128), jnp.float32),
        mesh=plsc.VectorSubcoreMesh(core_axis_name="c", subcore_axis_name="s"),
        scratch_types=[
            pltpu.VMEM((CH,), jnp.int32),
            pltpu.VMEM((CH,), jnp.int32),
            pltpu.VMEM((TAIL,), jnp.int32),
            pltpu.VMEM((CH, 128), jnp.float32),
            pltpu.VMEM((CH, 128), jnp.float32),
            pltpu.VMEM((TAIL, 128), jnp.float32),
            pltpu.VMEM_SHARED((NACC, 128), jnp.float32),
            pltpu.SemaphoreType.DMA,
            pltpu.SemaphoreType.DMA,
        ],
    )


def _tc_body(ea_ref, g_ref, sh_ref, ab_ref, w1_ref, b1_ref, w2_ref, b2_ref,
             red_ref, out_ref):
    x = ea_ref[...]
    h = jnp.maximum(
        jnp.dot(x, w1_ref[...], preferred_element_type=jnp.float32) + b1_ref[...], 0.0)
    tpw = jnp.dot(h, w2_ref[...], preferred_element_type=jnp.float32) + b2_ref[...]

    shab = jnp.dot(sh_ref[...], ab_ref[...], preferred_element_type=jnp.float32)

    g = g_ref[...]
    m = g * shab[:, 0:128]
    n = g * shab[:, 128:256]
    dot = m[:, 112:128] + n[:, 48:64] + n[:, 80:96]
    u0 = jnp.concatenate([m[:, 96:112], dot], 1)
    ux = m[:, 0:32]
    uy = m[:, 32:64]
    uz = m[:, 64:96]

    t0 = jnp.concatenate([u0] * 16, 1)
    tx = jnp.concatenate([ux] * 16, 1)
    ty = jnp.concatenate([uy] * 16, 1)
    tz = jnp.concatenate([uz] * 16, 1)
    w0 = tpw[:, 0:512]
    w1 = tpw[:, 512:1024]
    pall = jnp.concatenate([t0 * w0, tx * w1, ty * w1, tz * w1], 0)

    sred = jnp.dot(pall, red_ref[...], preferred_element_type=jnp.float32)

    ones_col = jnp.concatenate(
        [jnp.ones((BE, 16), jnp.float32), jnp.zeros((BE, 48), jnp.float32)], 1)
    out_ref[...] = jnp.concatenate(
        [sred[0:BE], sred[BE:2 * BE], sred[2 * BE:3 * BE], sred[3 * BE:4 * BE],
         ones_col], 1)


def _tc_fused(edge_attr, g, edge_sh, ab, w1, b1, w2p, b2p, red):
    return pl.pallas_call(
        _tc_body,
        grid=(NBLK,),
        in_specs=[
            pl.BlockSpec((BE, N_EDGE_FEAT), lambda i: (i, 0)),
            pl.BlockSpec((BE, 128), lambda i: (i, 0)),
            pl.BlockSpec((BE, 4), lambda i: (i, 0)),
            pl.BlockSpec((4, 256), lambda i: (0, 0)),
            pl.BlockSpec((N_EDGE_FEAT, HIDDEN), lambda i: (0, 0)),
            pl.BlockSpec((1, HIDDEN), lambda i: (0, 0)),
            pl.BlockSpec((HIDDEN, WEIGHT_NUMEL), lambda i: (0, 0)),
            pl.BlockSpec((1, WEIGHT_NUMEL), lambda i: (0, 0)),
            pl.BlockSpec((512, 16), lambda i: (0, 0)),
        ],
        out_specs=pl.BlockSpec((BE, 128), lambda i: (i, 0)),
        out_shape=jax.ShapeDtypeStruct((N_EDGES, 128), jnp.float32),
        compiler_params=pltpu.CompilerParams(
            dimension_semantics=("parallel",)),
    )(edge_attr, g, edge_sh, ab, w1, b1, w2p, b2p, red)


def _fin_body(a_ref, b_ref, n_ref, out_ref):
    a = a_ref[...]
    b = b_ref[...]
    tot = a[:, 0:64] + b[:, 0:64]
    cnt = a[:, 64:65] + b[:, 64:65]
    out_ref[...] = tot / jnp.maximum(cnt, 1.0) + n_ref[...]


def _fin_body2(acc_ref, n_ref, out_ref):
    a = acc_ref[0]
    b = acc_ref[1]
    tot = a[:, 0:64] + b[:, 0:64]
    cnt = a[:, 64:65] + b[:, 64:65]
    out_ref[...] = tot / jnp.maximum(cnt, 1.0) + n_ref[...]


def _finalize(acc, node_planes):
    return pl.pallas_call(
        _fin_body2,
        grid=(10,),
        in_specs=[
            pl.BlockSpec((2, 1000, 128), lambda i: (0, i, 0)),
            pl.BlockSpec((1000, IN_DIM), lambda i: (i, 0)),
        ],
        out_specs=pl.BlockSpec((1000, IN_DIM), lambda i: (i, 0)),
        out_shape=jax.ShapeDtypeStruct((N_NODES, IN_DIM), jnp.float32),
    )(acc, node_planes)


_Q = np.arange(512)
_PERM = np.concatenate([16 * (_Q % 32) + _Q // 32,
                        512 + 16 * (_Q % 32) + _Q // 32]).astype(np.int32)

_RED = np.zeros((512, 16), np.float32)
_RED[_Q, _Q // 32] = _INV_SQRT32

_AB = np.zeros((4, 256), np.float32)
for _g, (_s, _v) in enumerate([(1, 1.0), (0, 1.0), (2, 1.0), (0, 1.0),
                               (3, 1.0), (0, 1.0), (0, 1.0), (1, _INV_SQRT3)]):
    _AB[_s, 16 * _g:16 * (_g + 1)] = _v
_AB[2, 128 + 48:128 + 64] = _INV_SQRT3
_AB[3, 128 + 80:128 + 96] = _INV_SQRT3

_OFFS = np.minimum(np.arange(NCHUNK) * CH, TAIL_OFF)
_POS = (np.arange(NW)[:, None, None] * EPW
        + _OFFS[None, :, None]
        + np.arange(CH)[None, None, :]).astype(np.int32)


def kernel(node_attr, edge_index, edge_attr, edge_sh, fc_w1, fc_b1, fc_w2, fc_b2):
    src = edge_index[0]
    dst = edge_index[1]

    n1 = node_attr[:, 16:].reshape(N_NODES, MUL, 3)
    node_planes = jnp.concatenate(
        [node_attr[:, 0:16], n1[:, :, 0], n1[:, :, 1], n1[:, :, 2]], axis=1)
    n0 = node_attr[:, 0:16]
    table = jnp.concatenate(
        [n0, n1[:, :, 0], n0, n1[:, :, 1], n0, n1[:, :, 2], n0, n1[:, :, 0]],
        axis=1)


    w2p = fc_w2[:, _PERM]
    b2p = fc_b2[_PERM].reshape(1, WEIGHT_NUMEL)
    b1 = fc_b1.reshape(1, HIDDEN)

    g = _gather_call()(table, dst)
    tp_ext = _tc_fused(edge_attr, g, edge_sh, jnp.asarray(_AB), fc_w1, b1,
                       w2p, b2p, jnp.asarray(_RED))
    zeros = jnp.zeros((NACC, 128), jnp.float32)
    acc = _scatter_call()(tp_ext, src, zeros)
    planes_out = _finalize(acc, node_planes)

    o1 = planes_out[:, 16:].reshape(N_NODES, 3, MUL).transpose(0, 2, 1)
    return jnp.concatenate(
        [planes_out[:, 0:16], o1.reshape(N_NODES, 48)], axis=1)


# speedup vs baseline: 1.7341x; 1.0029x over previous
import functools

import jax
import jax.numpy as jnp
import numpy as np
from jax import lax
from jax.experimental import pallas as pl
from jax.experimental.pallas import tpu as pltpu
from jax.experimental.pallas import tpu_sc as plsc

N_NODES = 10000
N_EDGES = 160000
MUL = 16
IN_DIM = 64
N_EDGE_FEAT = 256
HIDDEN = 256
WEIGHT_NUMEL = 1024

NC = 2
NS = 16
NW = NC * NS
EPW = N_EDGES // NW
CH = 128
NCHUNK = (EPW + CH - 1) // CH
TAIL_OFF = EPW - CH
TAIL_NEW = EPW - (NCHUNK - 1) * CH
NACC = 10112
STRIPE = NACC // NS
TRASH = N_NODES

BE = 5000
NBLK = N_EDGES // BE

_INV_SQRT3 = 1.0 / np.sqrt(3.0)
_INV_SQRT32 = 1.0 / np.sqrt(32.0)


def _gather_body(table_hbm, dst_hbm, out_hbm, idx_v, r0, r1, sem0, sem1):
    c = lax.axis_index("c")
    s = lax.axis_index("s")
    wid = c * NS + s
    base = wid * EPW
    pltpu.sync_copy(dst_hbm.at[pl.ds(base, EPW)], idx_v)

    def ioff(j):
        return jnp.minimum(j * CH, TAIL_OFF)

    pltpu.async_copy(table_hbm.at[idx_v.at[pl.ds(0, CH)]], r0, sem0)

    def body(k, carry):
        j0 = 2 * k
        j1 = 2 * k + 1
        pltpu.async_copy(table_hbm.at[idx_v.at[pl.ds(ioff(j1), CH)]], r1, sem1)
        pltpu.make_async_copy(table_hbm.at[idx_v.at[pl.ds(0, CH)]], r0, sem0).wait()
        pltpu.sync_copy(r0, out_hbm.at[pl.ds(base + ioff(j0), CH)])

        @pl.when(j0 + 2 < NCHUNK)
        def _():
            pltpu.async_copy(
                table_hbm.at[idx_v.at[pl.ds(ioff(j0 + 2), CH)]], r0, sem0)

        pltpu.make_async_copy(table_hbm.at[idx_v.at[pl.ds(0, CH)]], r1, sem1).wait()
        pltpu.sync_copy(r1, out_hbm.at[pl.ds(base + ioff(j1), CH)])
        return carry

    lax.fori_loop(0, NCHUNK // 2, body, 0)


@functools.lru_cache(maxsize=None)
def _gather_call():
    return functools.partial(
        pl.kernel,
        _gather_body,
        out_type=jax.ShapeDtypeStruct((N_EDGES, 128), jnp.float32),
        mesh=plsc.VectorSubcoreMesh(core_axis_name="c", subcore_axis_name="s"),
        scratch_types=[
            pltpu.VMEM((EPW,), jnp.int32),
            pltpu.VMEM((CH, 128), jnp.float32),
            pltpu.VMEM((CH, 128), jnp.float32),
            pltpu.SemaphoreType.DMA,
            pltpu.SemaphoreType.DMA,
        ],
    )()


NFULL = EPW // CH
TAIL = EPW - NFULL * CH


def _scatter_body(tp_hbm, src_hbm, zeros_hbm, out_hbm, i0, i1, i8, r0, r1, r8,
                  acc, sem0, sem1):
    c = lax.axis_index("c")
    s = lax.axis_index("s")
    wid = c * NS + s
    base = wid * EPW

    @pl.when(s == 0)
    def _():
        pltpu.sync_copy(zeros_hbm, acc)

    plsc.subcore_barrier()

    pltpu.sync_copy(src_hbm.at[pl.ds(base, CH)], i0)
    pltpu.async_copy(tp_hbm.at[pl.ds(base, CH)], r0, sem0)

    def body(k, carry):
        j0 = 2 * k
        j1 = 2 * k + 1
        pltpu.sync_copy(src_hbm.at[pl.ds(base + j1 * CH, CH)], i1)
        pltpu.async_copy(tp_hbm.at[pl.ds(base + j1 * CH, CH)], r1, sem1)
        pltpu.make_async_copy(tp_hbm.at[pl.ds(base, CH)], r0, sem0).wait()
        pltpu.sync_copy(r0, acc.at[i0], add=True)

        @pl.when(j0 + 2 < NFULL)
        def _():
            pltpu.sync_copy(src_hbm.at[pl.ds(base + (j0 + 2) * CH, CH)], i0)
            pltpu.async_copy(tp_hbm.at[pl.ds(base + (j0 + 2) * CH, CH)], r0, sem0)

        pltpu.make_async_copy(tp_hbm.at[pl.ds(base, CH)], r1, sem1).wait()
        pltpu.sync_copy(r1, acc.at[i1], add=True)
        return carry

    lax.fori_loop(0, NFULL // 2, body, 0)
    pltpu.make_async_copy(tp_hbm.at[pl.ds(base, CH)], r0, sem0).wait()
    pltpu.sync_copy(r0, acc.at[i0], add=True)
    pltpu.sync_copy(src_hbm.at[pl.ds(base + NFULL * CH, TAIL)], i8)
    pltpu.sync_copy(tp_hbm.at[pl.ds(base + NFULL * CH, TAIL)], r8)
    pltpu.sync_copy(r8, acc.at[i8], add=True)

    plsc.subcore_barrier()
    pltpu.sync_copy(acc.at[pl.ds(s * STRIPE, STRIPE)],
                    out_hbm.at[c].at[pl.ds(s * STRIPE, STRIPE)])


@functools.lru_cache(maxsize=None)
def _scatter_call():
    return pl.kernel(
        _scatter_body,
        out_type=jax.ShapeDtypeStruct((NC, NACC, 128), jnp.float32),
        mesh=plsc.VectorSubcoreMesh(core_axis_name="c", subcore_axis_name="s"),
        scratch_types=[
            pltpu.VMEM((CH,), jnp.int32),
            pltpu.VMEM((CH,), jnp.int32),
            pltpu.VMEM((TAIL,), jnp.int32),
            pltpu.VMEM((CH, 128), jnp.float32),
            pltpu.VMEM((CH, 128), jnp.float32),
            pltpu.VMEM((TAIL, 128), jnp.float32),
            pltpu.VMEM_SHARED((NACC, 128), jnp.float32),
            pltpu.SemaphoreType.DMA,
            pltpu.SemaphoreType.DMA,
        ],
    )


def _tc_body(ea_ref, g_ref, sh_ref, ab_ref, w1_ref, b1_ref, w2_ref, b2_ref,
             red_ref, out_ref):
    x = ea_ref[...]
    h = jnp.maximum(
        jnp.dot(x, w1_ref[...], preferred_element_type=jnp.float32) + b1_ref[...], 0.0)
    tpw = jnp.dot(h, w2_ref[...], preferred_element_type=jnp.float32) + b2_ref[...]

    shab = jnp.dot(sh_ref[...], ab_ref[...], preferred_element_type=jnp.float32)

    g = g_ref[...]
    m = g * shab[:, 0:128]
    n = g * shab[:, 128:256]
    dot = m[:, 112:128] + n[:, 48:64] + n[:, 80:96]
    u0 = jnp.concatenate([m[:, 96:112], dot], 1)
    ux = m[:, 0:32]
    uy = m[:, 32:64]
    uz = m[:, 64:96]

    t0 = jnp.concatenate([u0] * 16, 1)
    tx = jnp.concatenate([ux] * 16, 1)
    ty = jnp.concatenate([uy] * 16, 1)
    tz = jnp.concatenate([uz] * 16, 1)
    w0 = tpw[:, 0:512]
    w1 = tpw[:, 512:1024]
    pall = jnp.concatenate([t0 * w0, tx * w1, ty * w1, tz * w1], 0)

    sred = jnp.dot(pall, red_ref[...], preferred_element_type=jnp.float32)

    ones_col = jnp.concatenate(
        [jnp.ones((BE, 16), jnp.float32), jnp.zeros((BE, 48), jnp.float32)], 1)
    out_ref[...] = jnp.concatenate(
        [sred[0:BE], sred[BE:2 * BE], sred[2 * BE:3 * BE], sred[3 * BE:4 * BE],
         ones_col], 1)


def _tc_fused(edge_attr, g, edge_sh, ab, w1, b1, w2p, b2p, red):
    return pl.pallas_call(
        _tc_body,
        grid=(NBLK,),
        in_specs=[
            pl.BlockSpec((BE, N_EDGE_FEAT), lambda i: (i, 0)),
            pl.BlockSpec((BE, 128), lambda i: (i, 0)),
            pl.BlockSpec((BE, 4), lambda i: (i, 0)),
            pl.BlockSpec((4, 256), lambda i: (0, 0)),
            pl.BlockSpec((N_EDGE_FEAT, HIDDEN), lambda i: (0, 0)),
            pl.BlockSpec((1, HIDDEN), lambda i: (0, 0)),
            pl.BlockSpec((HIDDEN, WEIGHT_NUMEL), lambda i: (0, 0)),
            pl.BlockSpec((1, WEIGHT_NUMEL), lambda i: (0, 0)),
            pl.BlockSpec((512, 16), lambda i: (0, 0)),
        ],
        out_specs=pl.BlockSpec((BE, 128), lambda i: (i, 0)),
        out_shape=jax.ShapeDtypeStruct((N_EDGES, 128), jnp.float32),
        compiler_params=pltpu.CompilerParams(
            dimension_semantics=("parallel",)),
    )(edge_attr, g, edge_sh, ab, w1, b1, w2p, b2p, red)


def _fin_body(a_ref, b_ref, n_ref, out_ref):
    a = a_ref[...]
    b = b_ref[...]
    tot = a[:, 0:64] + b[:, 0:64]
    cnt = a[:, 64:65] + b[:, 64:65]
    out_ref[...] = tot / jnp.maximum(cnt, 1.0) + n_ref[...]


def _fin_body2(acc_ref, n_ref, out_ref):
    a = acc_ref[0]
    b = acc_ref[1]
    tot = a[:, 0:64] + b[:, 0:64]
    cnt = a[:, 64:65] + b[:, 64:65]
    out_ref[...] = tot / jnp.maximum(cnt, 1.0) + n_ref[...]


def _finalize(acc, node_planes):
    return pl.pallas_call(
        _fin_body2,
        grid=(10,),
        in_specs=[
            pl.BlockSpec((2, 1000, 128), lambda i: (0, i, 0)),
            pl.BlockSpec((1000, IN_DIM), lambda i: (i, 0)),
        ],
        out_specs=pl.BlockSpec((1000, IN_DIM), lambda i: (i, 0)),
        out_shape=jax.ShapeDtypeStruct((N_NODES, IN_DIM), jnp.float32),
    )(acc, node_planes)


_Q = np.arange(512)
_PERM = np.concatenate([16 * (_Q % 32) + _Q // 32,
                        512 + 16 * (_Q % 32) + _Q // 32]).astype(np.int32)

_RED = np.zeros((512, 16), np.float32)
_RED[_Q, _Q // 32] = _INV_SQRT32

_AB = np.zeros((4, 256), np.float32)
for _g, (_s, _v) in enumerate([(1, 1.0), (0, 1.0), (2, 1.0), (0, 1.0),
                               (3, 1.0), (0, 1.0), (0, 1.0), (1, _INV_SQRT3)]):
    _AB[_s, 16 * _g:16 * (_g + 1)] = _v
_AB[2, 128 + 48:128 + 64] = _INV_SQRT3
_AB[3, 128 + 80:128 + 96] = _INV_SQRT3

_OFFS = np.minimum(np.arange(NCHUNK) * CH, TAIL_OFF)
_POS = (np.arange(NW)[:, None, None] * EPW
        + _OFFS[None, :, None]
        + np.arange(CH)[None, None, :]).astype(np.int32)


def kernel(node_attr, edge_index, edge_attr, edge_sh, fc_w1, fc_b1, fc_w2, fc_b2):
    src = edge_index[0]
    dst = edge_index[1]

    n1 = node_attr[:, 16:].reshape(N_NODES, MUL, 3)
    node_planes = jnp.concatenate(
        [node_attr[:, 0:16], n1[:, :, 0], n1[:, :, 1], n1[:, :, 2]], axis=1)
    n0 = node_attr[:, 0:16]
    table = jnp.concatenate(
        [n0, n1[:, :, 0], n0, n1[:, :, 1], n0, n1[:, :, 2], n0, n1[:, :, 0]],
        axis=1)


    w2p = fc_w2[:, _PERM]
    b2p = fc_b2[_PERM].reshape(1, WEIGHT_NUMEL)
    b1 = fc_b1.reshape(1, HIDDEN)

    g = _gather_call()(table, dst)
    tp_ext = _tc_fused(edge_attr, g, edge_sh, jnp.asarray(_AB), fc_w1, b1,
                       w2p, b2p, jnp.asarray(_RED))
    zeros = jnp.zeros((NACC, 128), jnp.float32)
    acc = _scatter_call()(tp_ext, src, zeros)
    planes_out = _finalize(acc, node_planes)

    o1 = planes_out[:, 16:].reshape(N_NODES, 3, MUL).transpose(0, 2, 1)
    return jnp.concatenate(
        [planes_out[:, 0:16], o1.reshape(N_NODES, 48)], axis=1)
